# baseline (speedup 1.0000x reference)
import functools

import jax
import jax.numpy as jnp
import numpy as np
from jax import lax
from jax.experimental import pallas as pl
from jax.experimental.pallas import tpu as pltpu
from jax.experimental.pallas import tpu_sc as plsc

N = 10000
E = 320000
D = 128
EPS = 1e-5

NC = 2
NS = 16
NW = NC * NS
CHUNK = 112
K = 93
EPT = K * CHUNK
EPAD = NW * EPT
GRP = CHUNK // 16
NDEN = 10240

_mesh = plsc.VectorSubcoreMesh(core_axis_name="c", subcore_axis_name="s")


@functools.partial(
    pl.kernel,
    out_type=(
        jax.ShapeDtypeStruct((NC, N, D), jnp.float32),
        jax.ShapeDtypeStruct((NC, NDEN), jnp.float32),
    ),
    mesh=_mesh,
    scratch_types=[
        [pltpu.VMEM((CHUNK, D), jnp.float32) for _ in range(3)],
        [pltpu.VMEM((CHUNK,), jnp.int32) for _ in range(3)],
        [pltpu.VMEM((CHUNK,), jnp.int32) for _ in range(3)],
        [pltpu.VMEM((CHUNK,), jnp.float32) for _ in range(3)],
        [pltpu.VMEM((CHUNK,), jnp.int32) for _ in range(3)],
        [pltpu.VMEM((CHUNK,), jnp.float32) for _ in range(3)],
        pltpu.VMEM((320,), jnp.float32),
        pltpu.VMEM_SHARED((N, D), jnp.float32),
        pltpu.VMEM_SHARED((NDEN,), jnp.float32),
        [pltpu.SemaphoreType.DMA for _ in range(3)],
        [pltpu.SemaphoreType.DMA for _ in range(3)],
        [pltpu.SemaphoreType.DMA for _ in range(3)],
        [pltpu.SemaphoreType.DMA for _ in range(3)],
        [pltpu.SemaphoreType.DMA for _ in range(3)],
        [pltpu.SemaphoreType.DMA for _ in range(3)],
    ],
)
def _sc_scatter(nr_hbm, src_hbm, dst_hbm, adv_hbm,
                out_s, out_d,
                rows, srcb, dstb, advb, dstc, advc, zden_v,
                accum, denom,
                sem_r, sem_w, sem_d, sem_src, sem_dst, sem_av):
    c = lax.axis_index("c")
    s = lax.axis_index("s")
    wid = c * NS + s
    zero16 = jnp.zeros((16,), jnp.float32)


    row0 = s * 640

    def zero_stripe(nrows):
        full, rem = divmod(nrows, CHUNK)
        for q in range(full):
            pltpu.sync_copy(rows[2], accum.at[pl.ds(row0 + q * CHUNK, CHUNK)])
        if rem:
            pltpu.sync_copy(rows[2].at[pl.ds(0, rem)],
                            accum.at[pl.ds(row0 + full * CHUNK, rem)])

    def _eoff(k):
        return pl.multiple_of(wid * EPT + k * CHUNK, 8)

    def start_idx(k, p):
        off = _eoff(k)
        pltpu.async_copy(src_hbm.at[pl.ds(off, CHUNK)], srcb[p], sem_src[p])
        pltpu.async_copy(dst_hbm.at[pl.ds(off, CHUNK)], dstb[p], sem_dst[p])
        pltpu.async_copy(adv_hbm.at[pl.ds(off, CHUNK)], advb[p], sem_av[p])

    def wait_idx(k, p):
        off = _eoff(k)
        pltpu.make_async_copy(src_hbm.at[pl.ds(off, CHUNK)], srcb[p], sem_src[p]).wait()
        pltpu.make_async_copy(dst_hbm.at[pl.ds(off, CHUNK)], dstb[p], sem_dst[p]).wait()
        pltpu.make_async_copy(adv_hbm.at[pl.ds(off, CHUNK)], advb[p], sem_av[p]).wait()

    def start_gather(k, p, r):
        pltpu.async_copy(nr_hbm.at[srcb[p]], rows[r], sem_r[r])

    def wait_gather(k, p, r):
        pltpu.make_async_copy(nr_hbm.at[srcb[p]], rows[r], sem_r[r]).wait()

    def start_scatter(f):
        pltpu.async_copy(rows[f], accum.at[dstc[f]], sem_w[f], add=True)

    def wait_scatter(f):
        pltpu.make_async_copy(rows[f], accum.at[dstc[f]], sem_w[f]).wait()

    def start_denom(f):
        pltpu.async_copy(advc[f], denom.at[dstc[f]], sem_d[f], add=True)

    def wait_denom(f):
        pltpu.make_async_copy(advc[f], denom.at[dstc[f]], sem_d[f]).wait()

    def scale_rows(f):
        def group_body(g, carry):
            base = g * 16
            av16 = advb[f][pl.ds(base, 16)]
            advc[f][pl.ds(base, 16)] = av16
            dstc[f][pl.ds(base, 16)] = dstb[f][pl.ds(base, 16)]
            for t in range(16):
                av = jnp.full((16,), av16[t], jnp.float32)
                i = base + t
                for j in range(8):
                    sl = pl.ds(j * 16, 16)
                    rows[f][i, sl] = rows[f][i, sl] * av
            return carry

        lax.fori_loop(0, GRP, group_body, 0, unroll=False)

    def process(k, f, w_den, w_scat, i_idx, i_gath):
        if w_den:
            wait_denom((f + 1) % 3)
        if i_idx:
            start_idx(k + 2, (f + 2) % 3)
        wait_gather(k, f, f)
        scale_rows(f)
        start_scatter(f)
        start_denom(f)
        if i_gath:
            if w_scat:
                wait_scatter((f + 2) % 3)
            wait_idx(k + 2, (f + 2) % 3)
            start_gather(k + 2, (f + 2) % 3, (f + 2) % 3)

    start_idx(0, 0)
    start_idx(1, 1)
    wait_idx(0, 0)
    start_gather(0, 0, 0)
    wait_idx(1, 1)
    start_gather(1, 1, 1)

    def zrow_body(i, carry):
        for j in range(8):
            rows[2][i, pl.ds(j * 16, 16)] = zero16
        return carry

    lax.fori_loop(0, CHUNK, zrow_body, 0, unroll=False)

    def zden_body(i, carry):
        zden_v[pl.ds(i * 16, 16)] = zero16
        return carry

    lax.fori_loop(0, 20, zden_body, 0, unroll=False)

    pltpu.sync_copy(zden_v, denom.at[pl.ds(s * 640, 320)])
    pltpu.sync_copy(zden_v, denom.at[pl.ds(s * 640 + 320, 320)])

    @pl.when(s < NS - 1)
    def _():
        zero_stripe(640)

    @pl.when(s == NS - 1)
    def _():
        zero_stripe(400)

    plsc.subcore_barrier()

    process(0, 0, False, False, True, True)
    process(1, 1, False, True, True, True)

    def triple_body(b, carry):
        k = 3 * b + 2
        process(k, 2, True, True, True, True)
        process(k + 1, 0, True, True, True, True)
        process(k + 2, 1, True, True, True, True)
        return carry

    lax.fori_loop(0, (K - 6) // 3, triple_body, 0, unroll=False)

    process(K - 4, (K - 4) % 3, True, True, True, True)
    process(K - 3, (K - 3) % 3, True, True, True, True)
    process(K - 2, (K - 2) % 3, True, True, False, False)
    process(K - 1, (K - 1) % 3, True, False, False, False)

    wait_denom((K - 2) % 3)
    wait_denom((K - 1) % 3)
    wait_scatter((K - 3) % 3)
    wait_scatter((K - 2) % 3)
    wait_scatter((K - 1) % 3)

    plsc.subcore_barrier()

    @pl.when(s < NS - 1)
    def _():
        pltpu.sync_copy(accum.at[pl.ds(row0, 640)], out_s.at[c, pl.ds(row0, 640)])

    @pl.when(s == NS - 1)
    def _():
        pltpu.sync_copy(accum.at[pl.ds(row0, 400)], out_s.at[c, pl.ds(row0, 400)])

    pltpu.sync_copy(denom.at[pl.ds(s * 640, 640)], out_d.at[c, pl.ds(s * 640, 640)])


_RB = 1000
_SQRT_HALF = np.float32(1.0 / np.sqrt(2.0))


def _proj_body(x_ref, w_ref, b_ref, o_ref):
    o_ref[...] = (
        jnp.dot(x_ref[...], w_ref[...], preferred_element_type=jnp.float32)
        + b_ref[...]
    )


def _proj(x, w, b):
    return pl.pallas_call(
        _proj_body,
        grid=(N // _RB,),
        in_specs=[
            pl.BlockSpec((_RB, D), lambda i: (i, 0)),
            pl.BlockSpec((D, D), lambda i: (0, 0)),
            pl.BlockSpec((1, D), lambda i: (0, 0)),
        ],
        out_specs=pl.BlockSpec((_RB, D), lambda i: (i, 0)),
        out_shape=jax.ShapeDtypeStruct((N, D), jnp.float32),
    )(x, w, b.reshape(1, D))


def _combine_math(sp_ref, dp_ref, nr_ref, w_ref, b_ref, g_ref, be_ref):
    ssum = sp_ref[0] + sp_ref[1]
    den = dp_ref[0] + dp_ref[1]
    aggr = jnp.where(den > 0.0, ssum / den, 0.0)
    h = (
        jnp.dot(aggr + nr_ref[...], w_ref[...], preferred_element_type=jnp.float32)
        + b_ref[...]
    )
    h = 0.5 * h * (1.0 + lax.erf(h * _SQRT_HALF))
    mu = jnp.mean(h, axis=-1, keepdims=True)
    var = jnp.mean((h - mu) ** 2, axis=-1, keepdims=True)
    return (h - mu) / jnp.sqrt(var + EPS) * g_ref[...] + be_ref[...]


def _combine_body(sp_ref, dp_ref, nr_ref, w_ref, b_ref, g_ref, be_ref, o_ref):
    o_ref[...] = _combine_math(sp_ref, dp_ref, nr_ref, w_ref, b_ref, g_ref, be_ref)


def _combine_out_body(sp_ref, dp_ref, nr_ref, w_ref, b_ref, g_ref, be_ref,
                      wo_ref, bo_ref, o_ref):
    ln = _combine_math(sp_ref, dp_ref, nr_ref, w_ref, b_ref, g_ref, be_ref)
    o_ref[...] = (
        jnp.dot(ln, wo_ref[...], preferred_element_type=jnp.float32) + bo_ref[...]
    )


_COMBINE_SPECS = [
    pl.BlockSpec((NC, _RB, D), lambda i: (0, i, 0)),
    pl.BlockSpec((NC, _RB, 1), lambda i: (0, i, 0)),
    pl.BlockSpec((_RB, D), lambda i: (i, 0)),
    pl.BlockSpec((D, D), lambda i: (0, 0)),
    pl.BlockSpec((1, D), lambda i: (0, 0)),
    pl.BlockSpec((1, D), lambda i: (0, 0)),
    pl.BlockSpec((1, D), lambda i: (0, 0)),
]


def _combine(sp, dp, nr, w, b, g, be):
    return pl.pallas_call(
        _combine_body,
        grid=(N // _RB,),
        in_specs=list(_COMBINE_SPECS),
        out_specs=pl.BlockSpec((_RB, D), lambda i: (i, 0)),
        out_shape=jax.ShapeDtypeStruct((N, D), jnp.float32),
    )(sp, dp.reshape(NC, NDEN, 1), nr, w, b.reshape(1, D), g.reshape(1, D),
      be.reshape(1, D))


def _combine_out(sp, dp, nr, w, b, g, be, wo, bo):
    return pl.pallas_call(
        _combine_out_body,
        grid=(N // _RB,),
        in_specs=list(_COMBINE_SPECS) + [
            pl.BlockSpec((D, D), lambda i: (0, 0)),
            pl.BlockSpec((1, D), lambda i: (0, 0)),
        ],
        out_specs=pl.BlockSpec((_RB, D), lambda i: (i, 0)),
        out_shape=jax.ShapeDtypeStruct((N, D), jnp.float32),
    )(sp, dp.reshape(NC, NDEN, 1), nr, w, b.reshape(1, D), g.reshape(1, D),
      be.reshape(1, D), wo, bo.reshape(1, D))


def kernel(node_attr, edge_index, batch_idx, adv_atts,
           W_in, b_in, W_l0, b_l0, g_l0, be_l0,
           W_l1, b_l1, g_l1, be_l1, W_out, b_out):
    src = edge_index[0]
    dst = edge_index[1]

    pad_e = EPAD - E
    pad_idx = (jnp.arange(pad_e, dtype=jnp.int32) % N).astype(jnp.int32)
    src_p = jnp.concatenate([src, pad_idx])
    dst_p = jnp.concatenate([dst, pad_idx])
    adv_p = jnp.concatenate([adv_atts, jnp.zeros((2, pad_e), jnp.float32)], axis=1)

    nr = _proj(node_attr, W_in, b_in)

    sp, dp = _sc_scatter(nr, src_p, dst_p, adv_p[0])
    nr = _combine(sp, dp, nr, W_l0, b_l0, g_l0, be_l0)

    sp, dp = _sc_scatter(nr, src_p, dst_p, adv_p[1])
    out = _combine_out(sp, dp, nr, W_l1, b_l1, g_l1, be_l1, W_out, b_out)

    return out


# speedup vs baseline: 1.0366x; 1.0366x over previous
import functools

import jax
import jax.numpy as jnp
import numpy as np
from jax import lax
from jax.experimental import pallas as pl
from jax.experimental.pallas import tpu as pltpu
from jax.experimental.pallas import tpu_sc as plsc

N = 10000
E = 320000
D = 128
EPS = 1e-5

NC = 2
NS = 16
NW = NC * NS
CHUNK = 96
K = 108
EPT = K * CHUNK
EPAD = NW * EPT
GRP = CHUNK // 16
NDEN = 10240

_mesh = plsc.VectorSubcoreMesh(core_axis_name="c", subcore_axis_name="s")


@functools.partial(
    pl.kernel,
    out_type=(
        jax.ShapeDtypeStruct((NC, N, D), jnp.float32),
        jax.ShapeDtypeStruct((NC, NDEN), jnp.float32),
    ),
    mesh=_mesh,
    scratch_types=[
        [pltpu.VMEM((CHUNK, D), jnp.float32) for _ in range(3)],
        [pltpu.VMEM((CHUNK,), jnp.int32) for _ in range(3)],
        [pltpu.VMEM((CHUNK,), jnp.int32) for _ in range(3)],
        [pltpu.VMEM((CHUNK,), jnp.float32) for _ in range(3)],
        [pltpu.VMEM((CHUNK,), jnp.int32) for _ in range(3)],
        [pltpu.VMEM((CHUNK,), jnp.float32) for _ in range(3)],
        pltpu.VMEM((320,), jnp.float32),
        pltpu.VMEM_SHARED((N, D), jnp.float32),
        pltpu.VMEM_SHARED((NDEN,), jnp.float32),
        [pltpu.SemaphoreType.DMA for _ in range(3)],
        [pltpu.SemaphoreType.DMA for _ in range(3)],
        [pltpu.SemaphoreType.DMA for _ in range(3)],
        [pltpu.SemaphoreType.DMA for _ in range(3)],
        [pltpu.SemaphoreType.DMA for _ in range(3)],
        [pltpu.SemaphoreType.DMA for _ in range(3)],
    ],
)
def _sc_scatter(nr_hbm, src_hbm, dst_hbm, adv_hbm,
                out_s, out_d,
                rows, srcb, dstb, advb, dstc, advc, zden_v,
                accum, denom,
                sem_r, sem_w, sem_d, sem_src, sem_dst, sem_av):
    c = lax.axis_index("c")
    s = lax.axis_index("s")
    wid = c * NS + s
    zero16 = jnp.zeros((16,), jnp.float32)


    row0 = s * 640

    def zero_stripe(nrows):
        full, rem = divmod(nrows, CHUNK)
        for q in range(full):
            pltpu.sync_copy(rows[2], accum.at[pl.ds(row0 + q * CHUNK, CHUNK)])
        if rem:
            pltpu.sync_copy(rows[2].at[pl.ds(0, rem)],
                            accum.at[pl.ds(row0 + full * CHUNK, rem)])

    def _eoff(k):
        return pl.multiple_of(wid * EPT + k * CHUNK, 8)

    def start_idx(k, p):
        off = _eoff(k)
        pltpu.async_copy(src_hbm.at[pl.ds(off, CHUNK)], srcb[p], sem_src[p])
        pltpu.async_copy(dst_hbm.at[pl.ds(off, CHUNK)], dstb[p], sem_dst[p])
        pltpu.async_copy(adv_hbm.at[pl.ds(off, CHUNK)], advb[p], sem_av[p])

    def wait_idx(k, p):
        off = _eoff(k)
        pltpu.make_async_copy(src_hbm.at[pl.ds(off, CHUNK)], srcb[p], sem_src[p]).wait()
        pltpu.make_async_copy(dst_hbm.at[pl.ds(off, CHUNK)], dstb[p], sem_dst[p]).wait()
        pltpu.make_async_copy(adv_hbm.at[pl.ds(off, CHUNK)], advb[p], sem_av[p]).wait()

    def start_gather(k, p, r):
        pltpu.async_copy(nr_hbm.at[srcb[p]], rows[r], sem_r[r])

    def wait_gather(k, p, r):
        pltpu.make_async_copy(nr_hbm.at[srcb[p]], rows[r], sem_r[r]).wait()

    def start_scatter(f):
        pltpu.async_copy(rows[f], accum.at[dstc[f]], sem_w[f], add=True)

    def wait_scatter(f):
        pltpu.make_async_copy(rows[f], accum.at[dstc[f]], sem_w[f]).wait()

    def start_denom(f):
        pltpu.async_copy(advc[f], denom.at[dstc[f]], sem_d[f], add=True)

    def wait_denom(f):
        pltpu.make_async_copy(advc[f], denom.at[dstc[f]], sem_d[f]).wait()

    def scale_rows(f):
        def group_body(g, carry):
            base = g * 16
            av16 = advb[f][pl.ds(base, 16)]
            advc[f][pl.ds(base, 16)] = av16
            dstc[f][pl.ds(base, 16)] = dstb[f][pl.ds(base, 16)]
            for t in range(16):
                av = jnp.full((16,), av16[t], jnp.float32)
                i = base + t
                for j in range(8):
                    sl = pl.ds(j * 16, 16)
                    rows[f][i, sl] = rows[f][i, sl] * av
            return carry

        lax.fori_loop(0, GRP, group_body, 0, unroll=False)

    def process(k, f, w_den, w_scat, i_idx, i_gath):
        if w_den:
            wait_denom((f + 1) % 3)
        if i_idx:
            start_idx(k + 2, (f + 2) % 3)
        wait_gather(k, f, f)
        scale_rows(f)
        start_scatter(f)
        start_denom(f)
        if i_gath:
            if w_scat:
                wait_scatter((f + 2) % 3)
            wait_idx(k + 2, (f + 2) % 3)
            start_gather(k + 2, (f + 2) % 3, (f + 2) % 3)

    start_idx(0, 0)
    start_idx(1, 1)
    wait_idx(0, 0)
    start_gather(0, 0, 0)
    wait_idx(1, 1)
    start_gather(1, 1, 1)

    def zrow_body(i, carry):
        for j in range(8):
            rows[2][i, pl.ds(j * 16, 16)] = zero16
        return carry

    lax.fori_loop(0, CHUNK, zrow_body, 0, unroll=False)

    def zden_body(i, carry):
        zden_v[pl.ds(i * 16, 16)] = zero16
        return carry

    lax.fori_loop(0, 20, zden_body, 0, unroll=False)

    pltpu.sync_copy(zden_v, denom.at[pl.ds(s * 640, 320)])
    pltpu.sync_copy(zden_v, denom.at[pl.ds(s * 640 + 320, 320)])

    @pl.when(s < NS - 1)
    def _():
        zero_stripe(640)

    @pl.when(s == NS - 1)
    def _():
        zero_stripe(400)

    plsc.subcore_barrier()

    process(0, 0, False, False, True, True)
    process(1, 1, False, True, True, True)

    def triple_body(b, carry):
        k = 3 * b + 2
        process(k, 2, True, True, True, True)
        process(k + 1, 0, True, True, True, True)
        process(k + 2, 1, True, True, True, True)
        return carry

    lax.fori_loop(0, (K - 6) // 3, triple_body, 0, unroll=False)

    process(K - 4, (K - 4) % 3, True, True, True, True)
    process(K - 3, (K - 3) % 3, True, True, True, True)
    process(K - 2, (K - 2) % 3, True, True, False, False)
    process(K - 1, (K - 1) % 3, True, False, False, False)

    wait_denom((K - 2) % 3)
    wait_denom((K - 1) % 3)
    wait_scatter((K - 3) % 3)
    wait_scatter((K - 2) % 3)
    wait_scatter((K - 1) % 3)

    plsc.subcore_barrier()

    @pl.when(s < NS - 1)
    def _():
        pltpu.sync_copy(accum.at[pl.ds(row0, 640)], out_s.at[c, pl.ds(row0, 640)])

    @pl.when(s == NS - 1)
    def _():
        pltpu.sync_copy(accum.at[pl.ds(row0, 400)], out_s.at[c, pl.ds(row0, 400)])

    pltpu.sync_copy(denom.at[pl.ds(s * 640, 640)], out_d.at[c, pl.ds(s * 640, 640)])


_RB = 1000
_SQRT_HALF = np.float32(1.0 / np.sqrt(2.0))


def _proj_body(x_ref, w_ref, b_ref, o_ref):
    o_ref[...] = (
        jnp.dot(x_ref[...], w_ref[...], preferred_element_type=jnp.float32)
        + b_ref[...]
    )


def _proj(x, w, b):
    return pl.pallas_call(
        _proj_body,
        grid=(N // _RB,),
        in_specs=[
            pl.BlockSpec((_RB, D), lambda i: (i, 0)),
            pl.BlockSpec((D, D), lambda i: (0, 0)),
            pl.BlockSpec((1, D), lambda i: (0, 0)),
        ],
        out_specs=pl.BlockSpec((_RB, D), lambda i: (i, 0)),
        out_shape=jax.ShapeDtypeStruct((N, D), jnp.float32),
    )(x, w, b.reshape(1, D))


def _combine_math(sp_ref, dp_ref, nr_ref, w_ref, b_ref, g_ref, be_ref):
    ssum = sp_ref[0] + sp_ref[1]
    den = dp_ref[0] + dp_ref[1]
    aggr = jnp.where(den > 0.0, ssum / den, 0.0)
    h = (
        jnp.dot(aggr + nr_ref[...], w_ref[...], preferred_element_type=jnp.float32)
        + b_ref[...]
    )
    h = 0.5 * h * (1.0 + lax.erf(h * _SQRT_HALF))
    mu = jnp.mean(h, axis=-1, keepdims=True)
    var = jnp.mean((h - mu) ** 2, axis=-1, keepdims=True)
    return (h - mu) / jnp.sqrt(var + EPS) * g_ref[...] + be_ref[...]


def _combine_body(sp_ref, dp_ref, nr_ref, w_ref, b_ref, g_ref, be_ref, o_ref):
    o_ref[...] = _combine_math(sp_ref, dp_ref, nr_ref, w_ref, b_ref, g_ref, be_ref)


def _combine_out_body(sp_ref, dp_ref, nr_ref, w_ref, b_ref, g_ref, be_ref,
                      wo_ref, bo_ref, o_ref):
    ln = _combine_math(sp_ref, dp_ref, nr_ref, w_ref, b_ref, g_ref, be_ref)
    o_ref[...] = (
        jnp.dot(ln, wo_ref[...], preferred_element_type=jnp.float32) + bo_ref[...]
    )


_COMBINE_SPECS = [
    pl.BlockSpec((NC, _RB, D), lambda i: (0, i, 0)),
    pl.BlockSpec((NC, _RB, 1), lambda i: (0, i, 0)),
    pl.BlockSpec((_RB, D), lambda i: (i, 0)),
    pl.BlockSpec((D, D), lambda i: (0, 0)),
    pl.BlockSpec((1, D), lambda i: (0, 0)),
    pl.BlockSpec((1, D), lambda i: (0, 0)),
    pl.BlockSpec((1, D), lambda i: (0, 0)),
]


def _combine(sp, dp, nr, w, b, g, be):
    return pl.pallas_call(
        _combine_body,
        grid=(N // _RB,),
        in_specs=list(_COMBINE_SPECS),
        out_specs=pl.BlockSpec((_RB, D), lambda i: (i, 0)),
        out_shape=jax.ShapeDtypeStruct((N, D), jnp.float32),
    )(sp, dp.reshape(NC, NDEN, 1), nr, w, b.reshape(1, D), g.reshape(1, D),
      be.reshape(1, D))


def _combine_out(sp, dp, nr, w, b, g, be, wo, bo):
    return pl.pallas_call(
        _combine_out_body,
        grid=(N // _RB,),
        in_specs=list(_COMBINE_SPECS) + [
            pl.BlockSpec((D, D), lambda i: (0, 0)),
            pl.BlockSpec((1, D), lambda i: (0, 0)),
        ],
        out_specs=pl.BlockSpec((_RB, D), lambda i: (i, 0)),
        out_shape=jax.ShapeDtypeStruct((N, D), jnp.float32),
    )(sp, dp.reshape(NC, NDEN, 1), nr, w, b.reshape(1, D), g.reshape(1, D),
      be.reshape(1, D), wo, bo.reshape(1, D))


def kernel(node_attr, edge_index, batch_idx, adv_atts,
           W_in, b_in, W_l0, b_l0, g_l0, be_l0,
           W_l1, b_l1, g_l1, be_l1, W_out, b_out):
    src = edge_index[0]
    dst = edge_index[1]

    pad_e = EPAD - E
    pad_idx = (jnp.arange(pad_e, dtype=jnp.int32) % N).astype(jnp.int32)
    src_p = jnp.concatenate([src, pad_idx])
    dst_p = jnp.concatenate([dst, pad_idx])
    adv_p = jnp.concatenate([adv_atts, jnp.zeros((2, pad_e), jnp.float32)], axis=1)

    nr = _proj(node_attr, W_in, b_in)

    sp, dp = _sc_scatter(nr, src_p, dst_p, adv_p[0])
    nr = _combine(sp, dp, nr, W_l0, b_l0, g_l0, be_l0)

    sp, dp = _sc_scatter(nr, src_p, dst_p, adv_p[1])
    out = _combine_out(sp, dp, nr, W_l1, b_l1, g_l1, be_l1, W_out, b_out)

    return out
